# Initial kernel scaffold; baseline (speedup 1.0000x reference)
import jax
import jax.numpy as jnp
from jax.experimental import pallas as pl


def kernel(x, edge_index, edge_attr, batch, x_emb1, x_emb2, edge_emb1s, edge_emb2s, W1s, b1s, W2s, b2s, gammas, betas, Wp, bp):
    raise NotImplementedError("write your pallas kernel here")


# speedup vs baseline: 2.5915x; 2.5915x over previous
import functools

import jax
import jax.numpy as jnp
from jax import lax
from jax.experimental import pallas as pl
from jax.experimental.pallas import tpu as pltpu
from jax.experimental.pallas import tpu_sc as plsc

NL = 5
EMB = 300
N = 10000
E = 160000
ET = E + N
NG = 128
FH = 160
FP = 2 * FH
NP = 10240
CH = 128
R = 1280
NBLK = NP // R
NS = 16
RPS = NP // NS

WSZS = [10640] * 13 + [10560] * 3
WST = [sum(WSZS[:i]) for i in range(16)]
WPAD = 10752
CHE = 64
GR = 24
NGRP = WPAD // CHE // GR
HD = 64

_mesh = plsc.VectorSubcoreMesh(core_axis_name="c", subcore_axis_name="s")
_f32 = jnp.float32


@functools.partial(
    pl.kernel,
    out_type=jax.ShapeDtypeStruct((2 * NP, FH), _f32),
    mesh=_mesh,
    compiler_params=pltpu.CompilerParams(use_tc_tiling_on_sc=False),
    scratch_types=[
        pltpu.VMEM((NP // CH // NS, CH), jnp.int32),
        pltpu.VMEM((CH, FH), _f32),
        pltpu.SemaphoreType.DMA,
    ],
)
def _h0(t12, idx, out, idx_v, buf_v, sem):
    c = lax.axis_index("c")
    s = lax.axis_index("s")
    npc = NP // CH // NS
    pltpu.sync_copy(idx.at[c * NS + s], idx_v)

    def body(j, _):
        pltpu.async_copy(t12.at[idx_v.at[j]], buf_v, sem).wait()
        pltpu.sync_copy(buf_v, out.at[pl.ds(c * NP + (s * npc + j) * CH, CH)])
        return 0

    lax.fori_loop(0, npc, body, 0)


@functools.partial(
    pl.kernel,
    out_type=jax.ShapeDtypeStruct((2 * NP, FH), _f32),
    mesh=_mesh,
    compiler_params=pltpu.CompilerParams(use_tc_tiling_on_sc=False),
    scratch_types=[
        pltpu.VMEM((GR, CHE), jnp.int32),
        pltpu.VMEM((GR, CHE), jnp.int32),
        pltpu.VMEM((GR, CHE), jnp.int32),
        pltpu.VMEM((2, CHE, FH), _f32),
        pltpu.VMEM((1, HD), jnp.int32),
        pltpu.VMEM((1, HD), jnp.int32),
        pltpu.VMEM((1,), jnp.int32),
        pltpu.VMEM((1, FH), _f32),
        pltpu.VMEM_SHARED((NP, FH), _f32),
        pltpu.SemaphoreType.DMA,
        pltpu.SemaphoreType.DMA,
        pltpu.SemaphoreType.DMA,
    ],
)
def _spmm(h2, src2, combo2, dstw, etab, hsrc, hcombo, hdst, zc, out,
          sidx_v, cidx_v, didx_v, buf_v, hs_v, hc_v, hd_v, lump_v, acc,
          sem0, sem1, semh):
    c = lax.axis_index("c")
    s = lax.axis_index("s")
    pltpu.sync_copy(zc.at[pl.ds(s * RPS, RPS)], acc.at[pl.ds(s * RPS, RPS)])
    plsc.subcore_barrier()
    sems = (sem0, sem1)

    def grp(g, _):
        pltpu.sync_copy(src2.at[(c * NS + s) * NGRP + g], sidx_v)
        pltpu.sync_copy(combo2.at[(c * NS + s) * NGRP + g], cidx_v)
        pltpu.sync_copy(dstw.at[s * NGRP + g], didx_v)
        pltpu.async_copy(etab.at[cidx_v.at[0]], buf_v.at[0], sems[0])
        for j in range(GR):
            slot = j % 2
            pltpu.make_async_copy(etab.at[cidx_v.at[j]], buf_v.at[slot],
                                  sems[slot]).wait()
            pltpu.async_copy(h2.at[sidx_v.at[j]], buf_v.at[slot], semh,
                             add=True).wait()
            if j + 1 < GR:
                pltpu.async_copy(etab.at[cidx_v.at[j + 1]],
                                 buf_v.at[1 - slot], sems[1 - slot])
            pltpu.sync_copy(buf_v.at[slot], acc.at[didx_v.at[j]], add=True)
        return 0

    lax.fori_loop(0, NGRP, grp, 0)
    plsc.subcore_barrier()

    pltpu.sync_copy(hsrc.at[c * NS + s], hs_v)
    pltpu.sync_copy(hcombo.at[c * NS + s], hc_v)
    pltpu.sync_copy(hdst.at[s], hd_v)
    pltpu.async_copy(etab.at[hc_v.at[0]], buf_v.at[0], sem0).wait()
    pltpu.async_copy(h2.at[hs_v.at[0]], buf_v.at[0], semh, add=True).wait()
    for k in range(FH // 16):
        lump_v[0, pl.ds(k * 16, 16)] = buf_v[0, 0, pl.ds(k * 16, 16)]

    def red(j, _):
        for k in range(FH // 16):
            lump_v[0, pl.ds(k * 16, 16)] = (lump_v[0, pl.ds(k * 16, 16)]
                                            + buf_v[0, j, pl.ds(k * 16, 16)])
        return 0

    lax.fori_loop(1, HD, red, 0)
    pltpu.sync_copy(lump_v, acc.at[hd_v], add=True)
    plsc.subcore_barrier()
    pltpu.sync_copy(acc.at[pl.ds(s * RPS, RPS)], out.at[pl.ds(c * NP + s * RPS, RPS)])


def _mlp_body(agg_ref, w1_ref, b1_ref, w2_ref, b2_ref, hpre_ref, s1_ref, s2_ref):
    i = pl.program_id(0)
    af = jnp.concatenate([agg_ref[0], agg_ref[1]], axis=1)
    z = jnp.dot(af, w1_ref[...], preferred_element_type=_f32) + b1_ref[...]
    z = jnp.maximum(z, 0.0)
    w2 = w2_ref[...]
    hp = (jnp.dot(z[:, :512], w2[:512], preferred_element_type=_f32)
          + jnp.dot(z[:, 512:], w2[512:], preferred_element_type=_f32)
          + b2_ref[...])
    rows = i * R + lax.broadcasted_iota(jnp.int32, (R, 1), 0)
    hp = jnp.where(rows < N, hp, 0.0)
    hpre_ref[0] = hp[:, :FH]
    hpre_ref[1] = hp[:, FH:]
    s1_ref[0] = jnp.sum(hp, axis=0, keepdims=True)
    s2_ref[0] = jnp.sum(hp * hp, axis=0, keepdims=True)


_mlp_call = pl.pallas_call(
    _mlp_body,
    grid=(NBLK,),
    in_specs=[
        pl.BlockSpec((2, R, FH), lambda i: (0, i, 0)),
        pl.BlockSpec((FP, 2 * EMB), lambda i: (0, 0)),
        pl.BlockSpec((1, 2 * EMB), lambda i: (0, 0)),
        pl.BlockSpec((2 * EMB, FP), lambda i: (0, 0)),
        pl.BlockSpec((1, FP), lambda i: (0, 0)),
    ],
    out_specs=[
        pl.BlockSpec((2, R, FH), lambda i: (0, i, 0)),
        pl.BlockSpec((1, 1, FP), lambda i: (i, 0, 0)),
        pl.BlockSpec((1, 1, FP), lambda i: (i, 0, 0)),
    ],
    out_shape=[
        jax.ShapeDtypeStruct((2, NP, FH), _f32),
        jax.ShapeDtypeStruct((NBLK, 1, FP), _f32),
        jax.ShapeDtypeStruct((NBLK, 1, FP), _f32),
    ],
)


def _bn_body(hp_ref, mean_ref, var_ref, g_ref, b_ref, out_ref, *, relu):
    i = pl.program_id(0)
    mean = mean_ref[...]
    var = var_ref[...]
    d = jnp.sqrt(var + 1e-5)
    g = g_ref[...]
    b = b_ref[...]
    h0 = (hp_ref[0] - mean[:, :FH]) / d[:, :FH] * g[:, :FH] + b[:, :FH]
    h1 = (hp_ref[1] - mean[:, FH:]) / d[:, FH:] * g[:, FH:] + b[:, FH:]
    if relu:
        h0 = jnp.maximum(h0, 0.0)
        h1 = jnp.maximum(h1, 0.0)
    rows = i * R + lax.broadcasted_iota(jnp.int32, (R, 1), 0)
    out_ref[0] = jnp.where(rows < N, h0, 0.0)
    out_ref[1] = jnp.where(rows < N, h1, 0.0)


def _make_bn_call(relu):
    return pl.pallas_call(
        functools.partial(_bn_body, relu=relu),
        grid=(NBLK,),
        in_specs=[
            pl.BlockSpec((2, R, FH), lambda i: (0, i, 0)),
            pl.BlockSpec((1, FP), lambda i: (0, 0)),
            pl.BlockSpec((1, FP), lambda i: (0, 0)),
            pl.BlockSpec((1, FP), lambda i: (0, 0)),
            pl.BlockSpec((1, FP), lambda i: (0, 0)),
        ],
        out_specs=pl.BlockSpec((2, R, FH), lambda i: (0, i, 0)),
        out_shape=jax.ShapeDtypeStruct((2, NP, FH), _f32),
    )


_bn_mid = _make_bn_call(True)
_bn_last = _make_bn_call(False)


def _final_body(h_ref, wp_ref, bp_ref, bt_ref, out_ref, gsum_ref, cnt_ref):
    i = pl.program_id(0)
    hf = jnp.concatenate([h_ref[0], h_ref[1]], axis=1)
    o = jnp.dot(hf, wp_ref[...], preferred_element_type=_f32) + bp_ref[...]
    nrm = jnp.sqrt(jnp.sum(o * o, axis=1, keepdims=True))
    nf = o / jnp.maximum(nrm, 1e-12)
    rows = i * R + lax.broadcasted_iota(jnp.int32, (R, NG), 0)
    g = lax.broadcasted_iota(jnp.int32, (R, NG), 1)
    p = jnp.where((bt_ref[...] == g) & (rows < N), 1.0, 0.0)
    ps = lax.dot_general(p, nf, (((0,), (0,)), ((), ())),
                         preferred_element_type=_f32,
                         precision=lax.Precision.HIGHEST)
    pc = lax.dot_general(p, jnp.ones((R, 8), _f32), (((0,), (0,)), ((), ())),
                         preferred_element_type=_f32,
                         precision=lax.Precision.HIGHEST)

    @pl.when(i == 0)
    def _():
        gsum_ref[...] = ps
        cnt_ref[...] = pc

    @pl.when(i > 0)
    def _():
        gsum_ref[...] += ps
        cnt_ref[...] += pc

    @pl.when(i == NBLK - 1)
    def _():
        gf = gsum_ref[...] / jnp.maximum(cnt_ref[...][:, :1], 1.0)
        nr = jnp.sqrt(jnp.sum(gf * gf, axis=1, keepdims=True))
        out_ref[...] = gf / jnp.maximum(nr, 1e-12)


_final_call = pl.pallas_call(
    _final_body,
    grid=(NBLK,),
    in_specs=[
        pl.BlockSpec((2, R, FH), lambda i: (0, i, 0)),
        pl.BlockSpec((FP, EMB), lambda i: (0, 0)),
        pl.BlockSpec((1, EMB), lambda i: (0, 0)),
        pl.BlockSpec((R, NG), lambda i: (i, 0)),
    ],
    out_specs=pl.BlockSpec((NG, EMB), lambda i: (0, 0)),
    out_shape=jax.ShapeDtypeStruct((NG, EMB), _f32),
    scratch_shapes=[
        pltpu.VMEM((NG, EMB), _f32),
        pltpu.VMEM((NG, 8), _f32),
    ],
)


def kernel(x, edge_index, edge_attr, batch, x_emb1, x_emb2, edge_emb1s,
           edge_emb2s, W1s, b1s, W2s, b2s, gammas, betas, Wp, bp):
    i32 = jnp.int32
    two = jnp.arange(2, dtype=i32)

    idx12 = jnp.pad(x[:, 0] * 3 + x[:, 1], (0, NP - N), constant_values=9)
    idx12_2 = (idx12[None, :] + (two * 16)[:, None]).reshape(
        2 * NS, NP // CH // NS, CH)

    ar = jnp.arange(N, dtype=edge_index.dtype)
    src_f = jnp.concatenate([edge_index[0], ar])
    dst_f = jnp.concatenate([edge_index[1], ar])
    combo_f = jnp.concatenate([edge_attr[:, 0] * 3 + edge_attr[:, 1],
                               jnp.full((N,), 12, dtype=i32)])
    order = jnp.argsort(dst_f, stable=True)
    src_s = src_f[order]
    dst_s = dst_f[order]
    combo_s = combo_f[order]
    srcw = jnp.stack([
        jnp.pad(src_s[st:st + sz], (0, WPAD - sz), constant_values=N)
        for st, sz in zip(WST, WSZS)])
    dstw = jnp.stack([
        jnp.pad(dst_s[st:st + sz], (0, WPAD - sz), constant_values=N)
        for st, sz in zip(WST, WSZS)])
    combow = jnp.stack([
        jnp.pad(combo_s[st:st + sz], (0, WPAD - sz), constant_values=15)
        for st, sz in zip(WST, WSZS)])
    r0 = dstw[:, 0]
    carry = jnp.concatenate([jnp.zeros((1,), bool),
                             dst_s[jnp.asarray(WST[1:]) - 1] == r0[1:]])
    ishead = (dstw == r0[:, None]) & carry[:, None]
    srcb = jnp.where(ishead, N, srcw)
    dstb = jnp.where(ishead, N, dstw)
    combob = jnp.where(ishead, 15, combow)
    src2 = (srcb[None] + (two * NP)[:, None, None]).reshape(
        2 * NS * NGRP, GR, CHE)
    combo2 = (combob[None] + (two * 16)[:, None, None]).reshape(
        2 * NS * NGRP, GR, CHE)
    dst3 = dstb.reshape(NS * NGRP, GR, CHE)
    hsrc = jnp.where(ishead[:, :HD], srcw[:, :HD], N)
    hcombo = jnp.where(ishead[:, :HD], combow[:, :HD], 15)
    hsrc2 = (hsrc[None] + (two * NP)[:, None, None]).reshape(2 * NS, 1, HD)
    hcombo2 = (hcombo[None] + (two * 16)[:, None, None]).reshape(2 * NS, 1, HD)
    hdst = jnp.where(carry, r0, N).astype(i32).reshape(NS, 1)
    batchb = jnp.broadcast_to(
        jnp.pad(batch, (0, NP - N))[:, None], (NP, NG)).astype(i32)

    t12 = (x_emb1[:3][:, None, :] + x_emb2[None, :, :]).reshape(9, EMB)
    t12p = jnp.zeros((16, FP), _f32).at[:9, :EMB].set(t12)
    t12h = jnp.concatenate([t12p[:, :FH], t12p[:, FH:]], axis=0)
    e9s = (edge_emb1s[:, :3][:, :, None, :]
           + edge_emb2s[:, None, :, :]).reshape(NL, 9, EMB)
    etabs = jnp.zeros((NL, 16, FP), _f32).at[:, :9, :EMB].set(e9s)
    etabs = etabs.at[:, 12, :EMB].set(edge_emb1s[:, 4] + edge_emb2s[:, 0])
    etabs = jnp.concatenate([etabs[:, :, :FH], etabs[:, :, FH:]], axis=1)
    w1p = jnp.zeros((NL, FP, 2 * EMB), _f32).at[:, :EMB].set(W1s)
    b1r = b1s[:, None, :]
    w2p = jnp.zeros((NL, 2 * EMB, FP), _f32).at[:, :, :EMB].set(W2s)
    b2r = jnp.pad(b2s, ((0, 0), (0, FP - EMB)))[:, None, :]
    gp = jnp.concatenate([gammas, jnp.ones((NL, FP - EMB), _f32)], axis=1)[:, None, :]
    btr = jnp.pad(betas, ((0, 0), (0, FP - EMB)))[:, None, :]
    wpp = jnp.zeros((FP, EMB), _f32).at[:EMB].set(Wp)
    bpr = bp[None, :]
    zc = jnp.zeros((NP, FH), _f32)

    h2 = _h0(t12h, idx12_2)

    for l in range(NL):
        agg2 = _spmm(h2, src2, combo2, dst3, etabs[l], hsrc2, hcombo2,
                     hdst, zc).reshape(2, NP, FH)
        hpre, s1, s2 = _mlp_call(agg2, w1p[l], b1r[l], w2p[l], b2r[l])
        hf = jnp.concatenate([hpre[0, :N], hpre[1, :N, :EMB - FH]], axis=1)
        mean = jnp.pad(jnp.mean(hf, axis=0), (0, FP - EMB))[None, :]
        var = jnp.pad(jnp.var(hf, axis=0), (0, FP - EMB))[None, :]
        bn = _bn_mid if l < NL - 1 else _bn_last
        h2 = bn(hpre, mean, var, gp[l], btr[l]).reshape(2 * NP, FH)

    return _final_call(h2.reshape(2, NP, FH), wpp, bpr, batchb)
